# baseline (speedup 1.0000x reference)
import functools

import jax
import jax.numpy as jnp
from jax import lax
from jax.experimental import pallas as pl
from jax.experimental.pallas import tpu as pltpu
from jax.experimental.pallas import tpu_sc as plsc

_STEPS = 1000
_EMBED = 128
_BATCH = 16384
_NC = 2
_NS = 16
_NW = _NC * _NS
_BPW = _BATCH // _NW
_CH = 128
_NCH = _BPW // _CH


@jax.jit
def _sc_embedding_gather(x, te_weight):
    mesh = plsc.VectorSubcoreMesh(core_axis_name="c", subcore_axis_name="s")

    @functools.partial(
        pl.kernel,
        mesh=mesh,
        out_type=jax.ShapeDtypeStruct((_BATCH, _EMBED), jnp.float32),
        scratch_types=[
            pltpu.VMEM((_BPW,), jnp.int32),
            pltpu.VMEM((_BPW, _EMBED), jnp.float32),
            pltpu.SemaphoreType.DMA,
            pltpu.SemaphoreType.DMA,
        ],
    )
    def k(idx_hbm, table_hbm, out_hbm, idx_v, rows_v, gsem, psem):
        wid = lax.axis_index("s") * _NC + lax.axis_index("c")
        base = wid * _BPW
        pltpu.sync_copy(idx_hbm.at[pl.ds(base, _BPW)], idx_v)
        gets = [
            pltpu.async_copy(
                table_hbm.at[idx_v.at[pl.ds(j * _CH, _CH)]],
                rows_v.at[pl.ds(j * _CH, _CH)],
                gsem,
            )
            for j in range(_NCH)
        ]
        half = _BPW // 2
        for g in gets[: _NCH // 2]:
            g.wait()
        put0 = pltpu.async_copy(
            rows_v.at[pl.ds(0, half)], out_hbm.at[pl.ds(base, half)], psem
        )
        for g in gets[_NCH // 2 :]:
            g.wait()
        pltpu.sync_copy(
            rows_v.at[pl.ds(half, half)], out_hbm.at[pl.ds(base + half, half)]
        )
        put0.wait()

    return k(x, te_weight)


def kernel(x, te_weight):
    out = _sc_embedding_gather(x, te_weight)
    return jnp.expand_dims(out, 1)


# speedup vs baseline: 1.1340x; 1.1340x over previous
import functools

import jax
import jax.numpy as jnp
from jax import lax
from jax.experimental import pallas as pl
from jax.experimental.pallas import tpu as pltpu
from jax.experimental.pallas import tpu_sc as plsc

_STEPS = 1000
_EMBED = 128
_BATCH = 16384
_NC = 2
_NS = 16
_NW = _NC * _NS
_BPW = _BATCH // _NW
_CH = 128
_NCH = _BPW // _CH


@jax.jit
def _sc_embedding_gather(x, te_weight):
    mesh = plsc.VectorSubcoreMesh(core_axis_name="c", subcore_axis_name="s")

    @functools.partial(
        pl.kernel,
        mesh=mesh,
        out_type=jax.ShapeDtypeStruct((_BATCH, _EMBED), jnp.float32),
        scratch_types=[
            pltpu.VMEM((_BPW,), jnp.int32),
            pltpu.VMEM((_BPW, _EMBED), jnp.float32),
            pltpu.VMEM_SHARED((_STEPS, _EMBED), jnp.float32),
            pltpu.SemaphoreType.DMA,
            pltpu.SemaphoreType.DMA,
        ],
    )
    def k(idx_hbm, table_hbm, out_hbm, idx_v, rows_v, table_sh, gsem, psem):
        sid = lax.axis_index("s")
        wid = sid * _NC + lax.axis_index("c")
        base = wid * _BPW
        iget = pltpu.async_copy(idx_hbm.at[pl.ds(base, _BPW)], idx_v, psem)

        @pl.when(sid == 0)
        def _():
            pltpu.sync_copy(table_hbm, table_sh)

        iget.wait()
        plsc.subcore_barrier()
        gets = [
            pltpu.async_copy(
                table_sh.at[idx_v.at[pl.ds(j * _CH, _CH)]],
                rows_v.at[pl.ds(j * _CH, _CH)],
                gsem,
            )
            for j in range(_NCH)
        ]
        for g in gets:
            g.wait()
        pltpu.sync_copy(rows_v, out_hbm.at[pl.ds(base, _BPW)])

    return k(x, te_weight)


def kernel(x, te_weight):
    out = _sc_embedding_gather(x, te_weight)
    return jnp.expand_dims(out, 1)


# speedup vs baseline: 1.1945x; 1.0534x over previous
import functools

import jax
import jax.numpy as jnp
from jax import lax
from jax.experimental import pallas as pl
from jax.experimental.pallas import tpu as pltpu
from jax.experimental.pallas import tpu_sc as plsc

_STEPS = 1000
_EMBED = 128
_BATCH = 16384
_NC = 2
_NS = 16
_NW = _NC * _NS
_BPW = _BATCH // _NW
_CH = 128
_NCH = _BPW // _CH


@jax.jit
def _sc_embedding_gather(x, te_weight):
    mesh = plsc.VectorSubcoreMesh(core_axis_name="c", subcore_axis_name="s")

    @functools.partial(
        pl.kernel,
        mesh=mesh,
        out_type=jax.ShapeDtypeStruct((_BATCH, _EMBED), jnp.float32),
        scratch_types=[
            pltpu.VMEM((_BPW,), jnp.int32),
            pltpu.VMEM((_BPW, _EMBED), jnp.float32),
            pltpu.VMEM_SHARED((_STEPS, _EMBED), jnp.float32),
            pltpu.SemaphoreType.DMA,
            pltpu.SemaphoreType.DMA,
        ],
    )
    def k(idx_hbm, table_hbm, out_hbm, idx_v, rows_v, table_sh, gsem, psem):
        sid = lax.axis_index("s")
        wid = sid * _NC + lax.axis_index("c")
        base = wid * _BPW
        iget = pltpu.async_copy(idx_hbm.at[pl.ds(base, _BPW)], idx_v, psem)

        @pl.when(sid == 0)
        def _():
            pltpu.sync_copy(table_hbm, table_sh)

        iget.wait()
        plsc.subcore_barrier()
        gets = [
            pltpu.async_copy(
                table_sh.at[idx_v.at[pl.ds(j * _CH, _CH)]],
                rows_v.at[pl.ds(j * _CH, _CH)],
                gsem,
            )
            for j in range(_NCH)
        ]
        puts = []
        for j in range(_NCH):
            gets[j].wait()
            puts.append(
                pltpu.async_copy(
                    rows_v.at[pl.ds(j * _CH, _CH)],
                    out_hbm.at[pl.ds(base + j * _CH, _CH)],
                    psem,
                )
            )
        for p in puts:
            p.wait()

    return k(x, te_weight)


def kernel(x, te_weight):
    out = _sc_embedding_gather(x, te_weight)
    return jnp.expand_dims(out, 1)
